# baseline (speedup 1.0000x reference)
import functools

import jax
import jax.numpy as jnp
from jax import lax
from jax.experimental import pallas as pl
from jax.experimental.pallas import tpu as pltpu
from jax.experimental.pallas import tpu_sc as plsc

VOCAB = 1000000
EMB = 64
B = 4096
L = 50

GRANULE = 128
N_GRAN = (B * L) // GRANULE
NW = 32
G_PER_W = N_GRAN // NW


def _make_gather():
    mesh = plsc.VectorSubcoreMesh(core_axis_name="c", subcore_axis_name="s")

    @functools.partial(
        pl.kernel,
        mesh=mesh,
        out_type=jax.ShapeDtypeStruct((B * L, EMB), jnp.float32),
        scratch_types=[
            pltpu.VMEM((G_PER_W, GRANULE), jnp.int32),
            pltpu.VMEM((GRANULE, EMB), jnp.float32),
            pltpu.SemaphoreType.DMA,
        ],
        compiler_params=pltpu.CompilerParams(use_tc_tiling_on_sc=False),
    )
    def gather_kernel(idx_hbm, table_hbm, out_hbm, idx_v, rows_v, sem):
        wid = lax.axis_index("s") * 2 + lax.axis_index("c")
        gbase = wid * G_PER_W
        pltpu.sync_copy(idx_hbm.at[wid], idx_v)

        def body(g, carry):
            pltpu.async_copy(table_hbm.at[idx_v.at[g]], rows_v, sem).wait()
            pltpu.sync_copy(
                rows_v, out_hbm.at[pl.ds((gbase + g) * GRANULE, GRANULE)]
            )
            return carry

        lax.fori_loop(0, G_PER_W, body, 0)

    return gather_kernel


_gather = _make_gather()


def kernel(input_variable, table):
    idx = input_variable.reshape(NW, G_PER_W, GRANULE).astype(jnp.int32)
    out = _gather(idx, table)
    return out.reshape(B, L, EMB)


# speedup vs baseline: 1.0448x; 1.0448x over previous
import functools

import jax
import jax.numpy as jnp
from jax import lax
from jax.experimental import pallas as pl
from jax.experimental.pallas import tpu as pltpu
from jax.experimental.pallas import tpu_sc as plsc

VOCAB = 1000000
EMB = 64
B = 4096
L = 50

GRANULE = 128
N_GRAN = (B * L) // GRANULE
NW = 32
G_PER_W = N_GRAN // NW
K = 5
N_GROUP = G_PER_W // K
GROUP_ROWS = K * GRANULE


def _make_gather():
    mesh = plsc.VectorSubcoreMesh(core_axis_name="c", subcore_axis_name="s")

    @functools.partial(
        pl.kernel,
        mesh=mesh,
        out_type=jax.ShapeDtypeStruct((B * L, EMB), jnp.float32),
        scratch_types=[
            pltpu.VMEM((G_PER_W, GRANULE), jnp.int32),
            pltpu.VMEM((2 * GROUP_ROWS, EMB), jnp.float32),
            pltpu.SemaphoreType.DMA,
            pltpu.SemaphoreType.DMA,
            pltpu.SemaphoreType.DMA,
        ],
        compiler_params=pltpu.CompilerParams(use_tc_tiling_on_sc=False),
    )
    def gather_kernel(idx_hbm, table_hbm, out_hbm, idx_v, rows_v, gsem, ssa, ssb):
        wid = lax.axis_index("s") * 2 + lax.axis_index("c")
        gbase = wid * G_PER_W
        pltpu.sync_copy(idx_hbm.at[wid], idx_v)

        def fire_gathers(g, set_):
            for j in range(K):
                pltpu.async_copy(
                    table_hbm.at[idx_v.at[g * K + j]],
                    rows_v.at[pl.ds((set_ * K + j) * GRANULE, GRANULE)],
                    gsem,
                )

        def wait_gathers(set_):
            pltpu.make_async_copy(
                out_hbm.at[pl.ds(0, GROUP_ROWS)],
                rows_v.at[pl.ds(set_ * GROUP_ROWS, GROUP_ROWS)],
                gsem,
            ).wait()

        def fire_store(g, set_, ssem):
            pltpu.async_copy(
                rows_v.at[pl.ds(set_ * GROUP_ROWS, GROUP_ROWS)],
                out_hbm.at[pl.ds((gbase + g * K) * GRANULE, GROUP_ROWS)],
                ssem,
            )

        def wait_store(g, set_, ssem):
            pltpu.make_async_copy(
                rows_v.at[pl.ds(set_ * GROUP_ROWS, GROUP_ROWS)],
                out_hbm.at[pl.ds((gbase + g * K) * GRANULE, GROUP_ROWS)],
                ssem,
            ).wait()

        fire_gathers(0, 0)
        wait_gathers(0)
        fire_store(0, 0, ssa)
        fire_gathers(1, 1)

        def body(p, carry):
            ga = 2 * p + 1
            gb = 2 * p + 2
            wait_gathers(1)
            fire_store(ga, 1, ssb)
            wait_store(ga - 1, 0, ssa)
            fire_gathers(gb, 0)
            wait_gathers(0)
            fire_store(gb, 0, ssa)
            wait_store(ga, 1, ssb)
            fire_gathers(gb + 1, 1)
            return carry

        lax.fori_loop(0, (N_GROUP - 2) // 2, body, 0)

        g_last = N_GROUP - 1
        wait_gathers(1)
        fire_store(g_last, 1, ssb)
        wait_store(g_last - 1, 0, ssa)
        wait_store(g_last, 1, ssb)

    return gather_kernel


_gather = _make_gather()


def kernel(input_variable, table):
    idx = input_variable.reshape(NW, G_PER_W, GRANULE).astype(jnp.int32)
    out = _gather(idx, table)
    return out.reshape(B, L, EMB)
